# baseline (speedup 1.0000x reference)
import jax
import jax.numpy as jnp
from jax import lax
from jax.experimental import pallas as pl
from jax.experimental.pallas import tpu as pltpu
from jax.experimental.pallas import tpu_sc as plsc

EMB_DIM = 64
SCALE = 8.0
SEQ = 50
BATCH = 16384

NUM_CORES = 2
NUM_SUBCORES = 16
NUM_WORKERS = NUM_CORES * NUM_SUBCORES

TOTAL_TOKENS = BATCH * SEQ
PER_WORKER = TOTAL_TOKENS // NUM_WORKERS
CHUNK = 128
NUM_CHUNKS = PER_WORKER // CHUNK
BBLOCKS = BATCH // CHUNK
NBUF = 2


def _body(tokens_hbm, table_hbm, out_hbm, tok_v, in_v, out_v, gsem, wsem):
    wid = lax.axis_index("s") * NUM_CORES + lax.axis_index("c")

    pltpu.sync_copy(tokens_hbm.at[wid], tok_v)

    lanes = lax.iota(jnp.int32, 16)

    def gather_start(j, b):
        pltpu.make_async_copy(
            table_hbm.at[tok_v.at[j]], in_v.at[b], gsem.at[b]
        ).start()

    def gather_wait(b):
        pltpu.make_async_copy(
            table_hbm.at[tok_v.at[0]], in_v.at[b], gsem.at[b]
        ).wait()

    def write_start(j, b):
        c = wid * NUM_CHUNKS + j
        t = c // BBLOCKS
        b0 = (c % BBLOCKS) * CHUNK
        pltpu.make_async_copy(
            out_v.at[b], out_hbm.at[t, :, pl.ds(b0, CHUNK)], wsem.at[b]
        ).start()

    def write_wait(b):
        pltpu.make_async_copy(
            out_v.at[b], out_hbm.at[0, :, pl.ds(0, CHUNK)], wsem.at[b]
        ).wait()

    def scale_transpose(b):
        for g in range(CHUNK // 16):
            rows = g * 16 + lanes
            sl = pl.ds(g * 16, 16)

            def feat_step(f, _):
                vals = plsc.load_gather(in_v.at[b], [rows, lax.broadcast(f, (16,))])
                out_v[b, f, sl] = vals * SCALE
                return 0

            lax.fori_loop(0, EMB_DIM, feat_step, 0, unroll=8)

    for b in range(NBUF):
        gather_start(b, b)

    def group_step(g, _):
        for b in range(NBUF):
            j = g * NBUF + b
            gather_wait(b)

            @pl.when(j >= NBUF)
            def _():
                write_wait(b)

            scale_transpose(b)
            write_start(j, b)

            @pl.when(j + NBUF < NUM_CHUNKS)
            def _():
                gather_start(j + NBUF, b)

        return 0

    lax.fori_loop(0, NUM_CHUNKS // NBUF, group_step, 0)

    for b in range(NBUF):
        write_wait(b)


@jax.jit
def _embed(tokens_grouped, table):
    mesh = plsc.VectorSubcoreMesh(core_axis_name="c", subcore_axis_name="s")
    out = pl.kernel(
        _body,
        out_type=jax.ShapeDtypeStruct((SEQ, EMB_DIM, BATCH), jnp.float32),
        mesh=mesh,
        scratch_types=[
            pltpu.VMEM((NUM_CHUNKS, CHUNK), jnp.int32),
            pltpu.VMEM((NBUF, CHUNK, EMB_DIM), jnp.float32),
            pltpu.VMEM((NBUF, EMB_DIM, CHUNK), jnp.float32),
            pltpu.SemaphoreType.DMA((NBUF,)),
            pltpu.SemaphoreType.DMA((NBUF,)),
        ],
        compiler_params=pltpu.CompilerParams(
            use_tc_tiling_on_sc=False, needs_layout_passes=False
        ),
    )(tokens_grouped, table)
    return out


def kernel(tokens, table):
    tokens_lin = tokens.astype(jnp.int32).T.reshape(-1)
    grouped = tokens_lin.reshape(NUM_WORKERS, NUM_CHUNKS, CHUNK)
    out = _embed(grouped, table)
    return jnp.transpose(out, (2, 0, 1))


# speedup vs baseline: 1.1695x; 1.1695x over previous
import jax
import jax.numpy as jnp
from jax import lax
from jax.experimental import pallas as pl
from jax.experimental.pallas import tpu as pltpu
from jax.experimental.pallas import tpu_sc as plsc

EMB_DIM = 64
SCALE = 8.0
SEQ = 50
BATCH = 16384

NUM_CORES = 2
NUM_SUBCORES = 16
NUM_WORKERS = NUM_CORES * NUM_SUBCORES

TOTAL_TOKENS = BATCH * SEQ
PER_WORKER = TOTAL_TOKENS // NUM_WORKERS
CHUNK = 128
NUM_CHUNKS = PER_WORKER // CHUNK
BBLOCKS = BATCH // CHUNK
NBUF = 2


def _body(tokens_hbm, table_hbm, out_hbm, tok_v, in_v, pad_v, out_v, gsem, wsem):
    wid = lax.axis_index("s") * NUM_CORES + lax.axis_index("c")

    pltpu.sync_copy(tokens_hbm.at[wid], tok_v)

    lanes = lax.iota(jnp.int32, 16)

    def gather_start(j, b):
        pltpu.make_async_copy(
            table_hbm.at[tok_v.at[j]], in_v.at[b], gsem.at[b]
        ).start()

    def gather_wait(b):
        pltpu.make_async_copy(
            table_hbm.at[tok_v.at[0]], in_v.at[b], gsem.at[b]
        ).wait()

    def write_start(j, b):
        c = wid * NUM_CHUNKS + j
        t = c // BBLOCKS
        b0 = (c % BBLOCKS) * CHUNK
        pltpu.make_async_copy(
            out_v.at[b], out_hbm.at[t, :, pl.ds(b0, CHUNK)], wsem.at[b]
        ).start()

    def write_wait(b):
        pltpu.make_async_copy(
            out_v.at[b], out_hbm.at[0, :, pl.ds(0, CHUNK)], wsem.at[b]
        ).wait()

    def scale_transpose(b):
        def copy_row(l, _):
            for c in range(EMB_DIM // 16):
                sl = pl.ds(c * 16, 16)
                pad_v[b, l, sl] = in_v[b, l, sl]
            return 0

        lax.fori_loop(0, CHUNK, copy_row, 0, unroll=4)

        for g in range(CHUNK // 16):
            rows = g * 16 + lanes
            sl = pl.ds(g * 16, 16)

            def feat_step(f, _):
                vals = plsc.load_gather(pad_v.at[b], [rows, lax.broadcast(f, (16,))])
                out_v[b, f, sl] = vals * SCALE
                return 0

            lax.fori_loop(0, EMB_DIM, feat_step, 0, unroll=8)

    for b in range(NBUF):
        gather_start(b, b)

    def group_step(g, _):
        for b in range(NBUF):
            j = g * NBUF + b
            gather_wait(b)

            @pl.when(j >= NBUF)
            def _():
                write_wait(b)

            scale_transpose(b)
            write_start(j, b)

            @pl.when(j + NBUF < NUM_CHUNKS)
            def _():
                gather_start(j + NBUF, b)

        return 0

    lax.fori_loop(0, NUM_CHUNKS // NBUF, group_step, 0)

    for b in range(NBUF):
        write_wait(b)


@jax.jit
def _embed(tokens_grouped, table):
    mesh = plsc.VectorSubcoreMesh(core_axis_name="c", subcore_axis_name="s")
    out = pl.kernel(
        _body,
        out_type=jax.ShapeDtypeStruct((SEQ, EMB_DIM, BATCH), jnp.float32),
        mesh=mesh,
        scratch_types=[
            pltpu.VMEM((NUM_CHUNKS, CHUNK), jnp.int32),
            pltpu.VMEM((NBUF, CHUNK, EMB_DIM), jnp.float32),
            pltpu.VMEM((NBUF, CHUNK, EMB_DIM + 1), jnp.float32),
            pltpu.VMEM((NBUF, EMB_DIM, CHUNK), jnp.float32),
            pltpu.SemaphoreType.DMA((NBUF,)),
            pltpu.SemaphoreType.DMA((NBUF,)),
        ],
        compiler_params=pltpu.CompilerParams(
            use_tc_tiling_on_sc=False, needs_layout_passes=False
        ),
    )(tokens_grouped, table)
    return out


def kernel(tokens, table):
    tokens_lin = tokens.astype(jnp.int32).T.reshape(-1)
    grouped = tokens_lin.reshape(NUM_WORKERS, NUM_CHUNKS, CHUNK)
    out = _embed(grouped, table)
    return jnp.transpose(out, (2, 0, 1))


# speedup vs baseline: 1.5835x; 1.3541x over previous
import jax
import jax.numpy as jnp
from jax import lax
from jax.experimental import pallas as pl
from jax.experimental.pallas import tpu as pltpu
from jax.experimental.pallas import tpu_sc as plsc

EMB_DIM = 64
SCALE = 8.0

NUM_CORES = 2
NUM_SUBCORES = 16
NUM_WORKERS = NUM_CORES * NUM_SUBCORES

TOTAL_TOKENS = 16384 * 50
PER_WORKER = TOTAL_TOKENS // NUM_WORKERS
CHUNK = 128
NUM_CHUNKS = PER_WORKER // CHUNK
NBUF = 4
NUM_GROUPS = NUM_CHUNKS // NBUF


def _body(tokens_hbm, table_hbm, out_hbm, idx_v, in_v, out_v, gsem, wsem):
    wid = lax.axis_index("s") * NUM_CORES + lax.axis_index("c")
    base = wid * PER_WORKER

    pltpu.sync_copy(tokens_hbm.at[wid], idx_v)

    def gather_start(j, b):
        pltpu.make_async_copy(
            table_hbm.at[idx_v.at[j]], in_v.at[b], gsem.at[b]
        ).start()

    def gather_wait(b):
        pltpu.make_async_copy(
            table_hbm.at[idx_v.at[0]], in_v.at[b], gsem.at[b]
        ).wait()

    def write_start(j, b):
        pltpu.make_async_copy(
            out_v.at[b], out_hbm.at[pl.ds(base + j * CHUNK, CHUNK)], wsem.at[b]
        ).start()

    def write_wait(b):
        pltpu.make_async_copy(
            out_v.at[b], out_hbm.at[pl.ds(base, CHUNK)], wsem.at[b]
        ).wait()

    def scale(b):
        def scale_row(i, _):
            for c in range(EMB_DIM // 16):
                sl = pl.ds(c * 16, 16)
                out_v[b, i, sl] = in_v[b, i, sl] * SCALE
            return 0

        lax.fori_loop(0, CHUNK, scale_row, 0, unroll=2)

    for b in range(NBUF):
        gather_start(b, b)

    for b in range(NBUF):
        gather_wait(b)
        scale(b)
        write_start(b, b)
        gather_start(NBUF + b, b)

    def group_step(g, _):
        for b in range(NBUF):
            j = g * NBUF + b
            gather_wait(b)
            write_wait(b)
            scale(b)
            write_start(j, b)
            gather_start(j + NBUF, b)
        return 0

    lax.fori_loop(1, NUM_GROUPS - 1, group_step, 0)

    for b in range(NBUF):
        j = (NUM_GROUPS - 1) * NBUF + b
        gather_wait(b)
        write_wait(b)
        scale(b)
        write_start(j, b)

    for b in range(NBUF):
        write_wait(b)


@jax.jit
def _embed(tokens_flat, table):
    mesh = plsc.VectorSubcoreMesh(core_axis_name="c", subcore_axis_name="s")
    grouped = tokens_flat.reshape(NUM_WORKERS, NUM_CHUNKS, CHUNK)
    out = pl.kernel(
        _body,
        out_type=jax.ShapeDtypeStruct((TOTAL_TOKENS, EMB_DIM), jnp.float32),
        mesh=mesh,
        scratch_types=[
            pltpu.VMEM((NUM_CHUNKS, CHUNK), jnp.int32),
            pltpu.VMEM((NBUF, CHUNK, EMB_DIM), jnp.float32),
            pltpu.VMEM((NBUF, CHUNK, EMB_DIM), jnp.float32),
            pltpu.SemaphoreType.DMA((NBUF,)),
            pltpu.SemaphoreType.DMA((NBUF,)),
        ],
        compiler_params=pltpu.CompilerParams(use_tc_tiling_on_sc=False),
    )(grouped, table)
    return out


def kernel(tokens, table):
    b, t = tokens.shape
    flat = tokens.reshape(-1).astype(jnp.int32)
    out = _embed(flat, table)
    return out.reshape(b, t, EMB_DIM)


# speedup vs baseline: 1.5850x; 1.0009x over previous
import jax
import jax.numpy as jnp
from jax import lax
from jax.experimental import pallas as pl
from jax.experimental.pallas import tpu as pltpu
from jax.experimental.pallas import tpu_sc as plsc

EMB_DIM = 64
SCALE = 8.0

NUM_CORES = 2
NUM_SUBCORES = 16
NUM_WORKERS = NUM_CORES * NUM_SUBCORES

TOTAL_TOKENS = 16384 * 50
PER_WORKER = TOTAL_TOKENS // NUM_WORKERS
CHUNK = 128
NUM_CHUNKS = PER_WORKER // CHUNK
NBUF = 4
NUM_GROUPS = NUM_CHUNKS // NBUF


def _body(tokens_hbm, table_hbm, out_hbm, idx_v, in_v, out_v, gsem, wsem):
    wid = lax.axis_index("s") * NUM_CORES + lax.axis_index("c")
    base = wid * PER_WORKER

    pltpu.sync_copy(tokens_hbm.at[wid], idx_v)

    def gather_start(j, b):
        pltpu.make_async_copy(
            table_hbm.at[idx_v.at[j]], in_v.at[b], gsem.at[b]
        ).start()

    def gather_wait(b):
        pltpu.make_async_copy(
            table_hbm.at[idx_v.at[0]], in_v.at[b], gsem.at[b]
        ).wait()

    def write_start(j, b):
        pltpu.make_async_copy(
            out_v.at[b],
            out_hbm.at[pl.ds(base // 2 + j * (CHUNK // 2), CHUNK // 2)],
            wsem.at[b],
        ).start()

    def write_wait(b):
        pltpu.make_async_copy(
            out_v.at[b], out_hbm.at[pl.ds(0, CHUNK // 2)], wsem.at[b]
        ).wait()

    def scale(b):
        def scale_row(i, _):
            for c in range(2 * EMB_DIM // 16):
                sl = pl.ds(c * 16, 16)
                src = pl.ds((c % 4) * 16, 16)
                out_v[b, i, sl] = in_v[b, 2 * i + c // 4, src] * SCALE
            return 0

        lax.fori_loop(0, CHUNK // 2, scale_row, 0, unroll=2)

    for b in range(NBUF):
        gather_start(b, b)

    for b in range(NBUF):
        gather_wait(b)
        scale(b)
        write_start(b, b)
        gather_start(NBUF + b, b)

    def group_step(g, _):
        for b in range(NBUF):
            j = g * NBUF + b
            gather_wait(b)
            write_wait(b)
            scale(b)
            write_start(j, b)
            gather_start(j + NBUF, b)
        return 0

    lax.fori_loop(1, NUM_GROUPS - 1, group_step, 0)

    for b in range(NBUF):
        j = (NUM_GROUPS - 1) * NBUF + b
        gather_wait(b)
        write_wait(b)
        scale(b)
        write_start(j, b)

    for b in range(NBUF):
        write_wait(b)


@jax.jit
def _embed(tokens_flat, table):
    mesh = plsc.VectorSubcoreMesh(core_axis_name="c", subcore_axis_name="s")
    grouped = tokens_flat.reshape(NUM_WORKERS, NUM_CHUNKS, CHUNK)
    out = pl.kernel(
        _body,
        out_type=jax.ShapeDtypeStruct((TOTAL_TOKENS // 2, 2 * EMB_DIM), jnp.float32),
        mesh=mesh,
        scratch_types=[
            pltpu.VMEM((NUM_CHUNKS, CHUNK), jnp.int32),
            pltpu.VMEM((NBUF, CHUNK, EMB_DIM), jnp.float32),
            pltpu.VMEM((NBUF, CHUNK // 2, 2 * EMB_DIM), jnp.float32),
            pltpu.SemaphoreType.DMA((NBUF,)),
            pltpu.SemaphoreType.DMA((NBUF,)),
        ],
        compiler_params=pltpu.CompilerParams(use_tc_tiling_on_sc=False),
    )(grouped, table)
    return out


def kernel(tokens, table):
    b, t = tokens.shape
    flat = tokens.reshape(-1).astype(jnp.int32)
    out = _embed(flat, table)
    return out.reshape(b, t, EMB_DIM)
